# baseline (speedup 1.0000x reference)
import functools
import jax
import jax.numpy as jnp
from jax import lax
from jax.experimental import pallas as pl
from jax.experimental.pallas import tpu as pltpu
from jax.experimental.pallas import tpu_sc as plsc

N = 100000
E = 6400000
G = 128

NC = 2
NS = 16
NW = NC * NS

NP = 100096
PAD = NP - N
RPW = NP // NW
RPS = NP // NS
GP = 136

EPC = E // NC
EPT = EPC // NS
EB = 1600
NSTEP = EPT // EB

VR = NP * 16 // 128

_mesh = plsc.VectorSubcoreMesh(core_axis_name="c", subcore_axis_name="s")
_sc_params = pltpu.CompilerParams(use_tc_tiling_on_sc=False)


def _f32(*shape):
  return jax.ShapeDtypeStruct(shape, jnp.float32)


@functools.partial(
    pl.kernel,
    out_type=_f32(NC, NP, 16),
    mesh=_mesh,
    compiler_params=_sc_params,
    scratch_types=[
        pltpu.VMEM((EB,), jnp.int32),
        pltpu.VMEM((EB, 16), jnp.float32),
        pltpu.VMEM_SHARED((NP, 16), jnp.float32),
    ],
)
def _sc_deg(dst_hbm, ones_hbm, zeros_hbm, out_hbm, idx_v, ones_v, acc_sh):
  c = lax.axis_index("c")
  s = lax.axis_index("s")
  row0 = s * RPS
  pltpu.sync_copy(zeros_hbm.at[pl.ds(row0, RPS)], acc_sh.at[pl.ds(row0, RPS)])
  pltpu.sync_copy(ones_hbm, ones_v)
  plsc.subcore_barrier()
  ebase = c * EPC + s * EPT

  def step(i, carry):
    pltpu.sync_copy(dst_hbm.at[pl.ds(ebase + i * EB, EB)], idx_v)
    pltpu.sync_copy(ones_v, acc_sh.at[idx_v], add=True)
    return carry

  lax.fori_loop(0, NSTEP, step, 0)
  plsc.subcore_barrier()
  pltpu.sync_copy(acc_sh.at[pl.ds(row0, RPS)],
                  out_hbm.at[c, pl.ds(row0, RPS)])


@functools.partial(
    pl.kernel,
    out_type=_f32(NC, NP, 16),
    mesh=_mesh,
    compiler_params=_sc_params,
    scratch_types=[
        pltpu.VMEM((EB,), jnp.int32),
        pltpu.VMEM((EB,), jnp.int32),
        pltpu.VMEM((EB, 16), jnp.float32),
        pltpu.VMEM_SHARED((NP, 16), jnp.float32),
    ],
)
def _prop(src_hbm, dst_hbm, g_hbm, zeros_hbm, out_hbm,
          idx_s, idx_d, rows_v, acc_sh):
  c = lax.axis_index("c")
  s = lax.axis_index("s")
  row0 = s * RPS
  pltpu.sync_copy(zeros_hbm.at[pl.ds(row0, RPS)],
                  acc_sh.at[pl.ds(row0, RPS)])
  plsc.subcore_barrier()
  ebase = c * EPC + s * EPT

  def step(i, carry):
    base = ebase + i * EB
    pltpu.sync_copy(src_hbm.at[pl.ds(base, EB)], idx_s)
    pltpu.sync_copy(dst_hbm.at[pl.ds(base, EB)], idx_d)
    pltpu.sync_copy(g_hbm.at[idx_s], rows_v)
    pltpu.sync_copy(rows_v, acc_sh.at[idx_d], add=True)
    return carry

  lax.fori_loop(0, NSTEP, step, 0)
  plsc.subcore_barrier()
  pltpu.sync_copy(acc_sh.at[pl.ds(row0, RPS)],
                  out_hbm.at[c, pl.ds(row0, RPS)])


@functools.partial(
    pl.kernel,
    out_type=_f32(NC, GP, 16),
    mesh=_mesh,
    compiler_params=_sc_params,
    scratch_types=[
        pltpu.VMEM((RPW,), jnp.int32),
        pltpu.VMEM((RPW, 16), jnp.float32),
        pltpu.VMEM_SHARED((GP, 16), jnp.float32),
    ],
)
def _sc_pool(h_hbm, batch_hbm, zeros_hbm, out_hbm, idx_v, rows_v, acc_sh):
  c = lax.axis_index("c")
  s = lax.axis_index("s")

  @pl.when(s == 0)
  def _():
    pltpu.sync_copy(zeros_hbm, acc_sh)

  plsc.subcore_barrier()
  row0 = (c * NS + s) * RPW
  pltpu.sync_copy(h_hbm.at[pl.ds(row0, RPW)], rows_v)
  pltpu.sync_copy(batch_hbm.at[pl.ds(row0, RPW)], idx_v)
  pltpu.sync_copy(rows_v, acc_sh.at[idx_v], add=True)
  plsc.subcore_barrier()

  @pl.when(s == 0)
  def _():
    pltpu.sync_copy(acc_sh, out_hbm.at[c])


_TCR = 3128
_TCG = VR // _TCR


def _vspec():
  return pl.BlockSpec((_TCR, 128), lambda i: (i, 0))


def _full_spec(r, f):
  return pl.BlockSpec((r, f), lambda i: (0, 0))


def _tc1_body(d0, d1, x, dinv_o, gx_o):
  dinv = lax.rsqrt(d0[...] + d1[...] + 1.0)
  dinv_o[...] = dinv
  gx_o[...] = x[...] * dinv


def _tc1(d0, d1, x):
  return pl.pallas_call(
      _tc1_body,
      grid=(_TCG,),
      in_specs=[_vspec(), _vspec(), _vspec()],
      out_specs=[_vspec(), _vspec()],
      out_shape=[_f32(VR, 128), _f32(VR, 128)],
  )(d0, d1, x)


def _tc2_body(a0, a1, gx, dinv, BW1, b1t, BW2, g2_o):
  p = dinv[...] * (a0[...] + a1[...] + gx[...])
  h1 = jnp.maximum(
      jnp.dot(p, BW1[...], preferred_element_type=jnp.float32) + b1t[...],
      0.0)
  g2_o[...] = dinv[...] * jnp.dot(h1, BW2[...],
                                  preferred_element_type=jnp.float32)


def _tc2(a0, a1, gx, dinv, BW1, b1t, BW2):
  return pl.pallas_call(
      _tc2_body,
      grid=(_TCG,),
      in_specs=[_vspec(), _vspec(), _vspec(), _vspec(),
                _full_spec(128, 256), _full_spec(1, 256),
                _full_spec(256, 128)],
      out_specs=_vspec(),
      out_shape=_f32(VR, 128),
  )(a0, a1, gx, dinv, BW1, b1t, BW2)


def _tc3_body(a0, a1, g2, dinv, b2t, BW3, g3_o):
  h2 = jnp.maximum(dinv[...] * (a0[...] + a1[...] + g2[...]) + b2t[...], 0.0)
  g3_o[...] = dinv[...] * jnp.dot(h2, BW3[...],
                                  preferred_element_type=jnp.float32)


def _tc3(a0, a1, g2, dinv, b2t, BW3):
  return pl.pallas_call(
      _tc3_body,
      grid=(_TCG,),
      in_specs=[_vspec(), _vspec(), _vspec(), _vspec(),
                _full_spec(1, 128), _full_spec(128, 128)],
      out_specs=_vspec(),
      out_shape=_f32(VR, 128),
  )(a0, a1, g2, dinv, b2t, BW3)


def _tc4_body(a0, a1, g3, dinv, b3t, e8t, h_o):
  h3 = jnp.maximum(dinv[...] * (a0[...] + a1[...] + g3[...]) + b3t[...], 0.0)
  h_o[...] = h3 + e8t[...]


def _tc4(a0, a1, g3, dinv, b3t, e8t):
  return pl.pallas_call(
      _tc4_body,
      grid=(_TCG,),
      in_specs=[_vspec(), _vspec(), _vspec(), _vspec(),
                _full_spec(1, 128), _full_spec(1, 128)],
      out_specs=_vspec(),
      out_shape=_f32(VR, 128),
  )(a0, a1, g3, dinv, b3t, e8t)


def _tc5_body(s0, s1, Wfc, bfc, out_o):
  acc = (s0[...] + s1[...])[:G]
  sums = acc[:, :8]
  cnts = jnp.maximum(acc[:, 8:9], 1.0)
  pooled = sums / cnts
  out_o[...] = jnp.dot(pooled, Wfc[...],
                       preferred_element_type=jnp.float32) + bfc[...]


def _tc5(s0, s1, Wfc, bfc):
  return pl.pallas_call(
      _tc5_body,
      out_shape=_f32(G, 3),
  )(s0, s1, Wfc, bfc)


def _view(a):
  return a.reshape(VR, 128)


def _unview(a):
  return a.reshape(NP, 16)


@jax.jit
def kernel(x, edge_index, batch, W1, b1, W2, b2, W3, b3, Wfc, bfc):
  src = edge_index[0]
  dst = edge_index[1]
  f32 = jnp.float32
  x16 = jnp.pad(x, ((0, PAD), (0, 13)))
  batch_p = jnp.pad(batch, (0, PAD), constant_values=G)

  eye8 = jnp.eye(8, dtype=f32)
  BW1 = jnp.kron(eye8, jnp.pad(W1, ((0, 13), (0, 0))))
  BW2 = jnp.kron(eye8, W2)
  BW3 = jnp.kron(eye8, jnp.pad(W3, ((0, 0), (0, 8))))
  b1t = jnp.tile(b1, 8).reshape(1, 256)
  b2t = jnp.tile(b2, 8).reshape(1, 128)
  b3t = jnp.tile(jnp.pad(b3, (0, 8)), 8).reshape(1, 128)
  e8t = jnp.tile(jnp.zeros((16,), f32).at[8].set(1.0), 8).reshape(1, 128)

  ones_eb = jnp.ones((EB, 16), f32)
  zeros16 = jnp.zeros((NP, 16), f32)
  zgp = jnp.zeros((GP, 16), f32)

  deg_pp = _sc_deg(dst, ones_eb, zeros16)
  dinv, gx = _tc1(_view(deg_pp[0]), _view(deg_pp[1]), _view(x16))

  aggx = _prop(src, dst, _unview(gx), zeros16)
  g2 = _tc2(_view(aggx[0]), _view(aggx[1]), gx, dinv, BW1, b1t, BW2)

  agg2 = _prop(src, dst, _unview(g2), zeros16)
  g3 = _tc3(_view(agg2[0]), _view(agg2[1]), g2, dinv, b2t, BW3)

  agg3 = _prop(src, dst, _unview(g3), zeros16)
  h16 = _tc4(_view(agg3[0]), _view(agg3[1]), g3, dinv, b3t, e8t)

  sums_pp = _sc_pool(_unview(h16), batch_p, zgp)
  out = _tc5(sums_pp[0], sums_pp[1], Wfc, bfc.reshape(1, -1))
  return out


# speedup vs baseline: 1.2176x; 1.2176x over previous
import functools
import jax
import jax.numpy as jnp
from jax import lax
from jax.experimental import pallas as pl
from jax.experimental.pallas import tpu as pltpu
from jax.experimental.pallas import tpu_sc as plsc

N = 100000
E = 6400000
G = 128

NC = 2
NS = 16
NW = NC * NS

NP = 100096
PAD = NP - N
RPW = NP // NW
RPS = NP // NS
GP = 136

EPC = E // NC
EPT = EPC // NS
EB = 1000
NSTEP = EPT // EB
PB = 800
PSTEP = EPT // PB

VR = NP * 16 // 128

_mesh = plsc.VectorSubcoreMesh(core_axis_name="c", subcore_axis_name="s")
_sc_params = pltpu.CompilerParams(use_tc_tiling_on_sc=False)


def _f32(*shape):
  return jax.ShapeDtypeStruct(shape, jnp.float32)


@functools.partial(
    pl.kernel,
    out_type=_f32(NC, NP, 16),
    mesh=_mesh,
    compiler_params=_sc_params,
    scratch_types=[
        pltpu.VMEM((EB,), jnp.int32),
        pltpu.VMEM((EB,), jnp.int32),
        pltpu.VMEM((EB, 16), jnp.float32),
        pltpu.SemaphoreType.DMA,
        pltpu.SemaphoreType.DMA,
        pltpu.VMEM_SHARED((NP, 16), jnp.float32),
    ],
)
def _sc_deg(dst_hbm, ones_hbm, zeros_hbm, out_hbm,
            idx0, idx1, ones_v, sem0, sem1, acc_sh):
  c = lax.axis_index("c")
  s = lax.axis_index("s")
  row0 = s * RPS
  pltpu.sync_copy(zeros_hbm.at[pl.ds(row0, RPS)], acc_sh.at[pl.ds(row0, RPS)])
  pltpu.sync_copy(ones_hbm, ones_v)
  plsc.subcore_barrier()
  ebase = c * EPC + s * EPT

  def _start(i, idx, sem):
    pltpu.async_copy(dst_hbm.at[pl.ds(ebase + i * EB, EB)], idx, sem)

  def _drain(i, idx, sem):
    pltpu.make_async_copy(dst_hbm.at[pl.ds(ebase + i * EB, EB)], idx,
                          sem).wait()
    pltpu.sync_copy(ones_v, acc_sh.at[idx], add=True)

  _start(0, idx0, sem0)

  def step(k, carry):
    i0 = 2 * k
    _start(i0 + 1, idx1, sem1)
    _drain(i0, idx0, sem0)

    @pl.when(i0 + 2 < NSTEP)
    def _():
      _start(i0 + 2, idx0, sem0)

    _drain(i0 + 1, idx1, sem1)
    return carry

  lax.fori_loop(0, NSTEP // 2, step, 0)
  plsc.subcore_barrier()
  pltpu.sync_copy(acc_sh.at[pl.ds(row0, RPS)],
                  out_hbm.at[c, pl.ds(row0, RPS)])


@functools.partial(
    pl.kernel,
    out_type=_f32(NC, NP, 16),
    mesh=_mesh,
    compiler_params=_sc_params,
    scratch_types=[
        pltpu.VMEM((PB,), jnp.int32),
        pltpu.VMEM((PB,), jnp.int32),
        pltpu.VMEM((PB,), jnp.int32),
        pltpu.VMEM((PB,), jnp.int32),
        pltpu.VMEM((PB, 16), jnp.float32),
        pltpu.VMEM((PB, 16), jnp.float32),
        pltpu.SemaphoreType.DMA,
        pltpu.SemaphoreType.DMA,
        pltpu.VMEM_SHARED((NP, 16), jnp.float32),
    ],
)
def _prop(src_hbm, dst_hbm, g_hbm, zeros_hbm, out_hbm,
          ixs0, ixd0, ixs1, ixd1, rows0, rows1, sem0, sem1, acc_sh):
  c = lax.axis_index("c")
  s = lax.axis_index("s")
  row0 = s * RPS
  pltpu.sync_copy(zeros_hbm.at[pl.ds(row0, RPS)],
                  acc_sh.at[pl.ds(row0, RPS)])
  plsc.subcore_barrier()
  ebase = c * EPC + s * EPT

  def _start(i, ixs, ixd, rows, sem):
    base = ebase + i * PB
    pltpu.sync_copy(src_hbm.at[pl.ds(base, PB)], ixs)
    pltpu.sync_copy(dst_hbm.at[pl.ds(base, PB)], ixd)
    pltpu.async_copy(g_hbm.at[ixs], rows, sem)

  def _drain(ixs, ixd, rows, sem):
    pltpu.make_async_copy(g_hbm.at[ixs], rows, sem).wait()
    pltpu.sync_copy(rows, acc_sh.at[ixd], add=True)

  _start(0, ixs0, ixd0, rows0, sem0)

  def step(k, carry):
    i0 = 2 * k
    _start(i0 + 1, ixs1, ixd1, rows1, sem1)
    _drain(ixs0, ixd0, rows0, sem0)

    @pl.when(i0 + 2 < PSTEP)
    def _():
      _start(i0 + 2, ixs0, ixd0, rows0, sem0)

    _drain(ixs1, ixd1, rows1, sem1)
    return carry

  lax.fori_loop(0, PSTEP // 2, step, 0)
  plsc.subcore_barrier()
  pltpu.sync_copy(acc_sh.at[pl.ds(row0, RPS)],
                  out_hbm.at[c, pl.ds(row0, RPS)])


@functools.partial(
    pl.kernel,
    out_type=_f32(NC, GP, 16),
    mesh=_mesh,
    compiler_params=_sc_params,
    scratch_types=[
        pltpu.VMEM((RPW,), jnp.int32),
        pltpu.VMEM((RPW, 16), jnp.float32),
        pltpu.VMEM_SHARED((GP, 16), jnp.float32),
    ],
)
def _sc_pool(h_hbm, batch_hbm, zeros_hbm, out_hbm, idx_v, rows_v, acc_sh):
  c = lax.axis_index("c")
  s = lax.axis_index("s")

  @pl.when(s == 0)
  def _():
    pltpu.sync_copy(zeros_hbm, acc_sh)

  plsc.subcore_barrier()
  row0 = (c * NS + s) * RPW
  pltpu.sync_copy(h_hbm.at[pl.ds(row0, RPW)], rows_v)
  pltpu.sync_copy(batch_hbm.at[pl.ds(row0, RPW)], idx_v)
  pltpu.sync_copy(rows_v, acc_sh.at[idx_v], add=True)
  plsc.subcore_barrier()

  @pl.when(s == 0)
  def _():
    pltpu.sync_copy(acc_sh, out_hbm.at[c])


_TCR = 3128
_TCG = VR // _TCR


def _vspec():
  return pl.BlockSpec((_TCR, 128), lambda i: (i, 0))


def _full_spec(r, f):
  return pl.BlockSpec((r, f), lambda i: (0, 0))


def _tc1_body(d0, d1, x, dinv_o, gx_o):
  dinv = lax.rsqrt(d0[...] + d1[...] + 1.0)
  dinv_o[...] = dinv
  gx_o[...] = x[...] * dinv


def _tc1(d0, d1, x):
  return pl.pallas_call(
      _tc1_body,
      grid=(_TCG,),
      in_specs=[_vspec(), _vspec(), _vspec()],
      out_specs=[_vspec(), _vspec()],
      out_shape=[_f32(VR, 128), _f32(VR, 128)],
  )(d0, d1, x)


def _tc2_body(a0, a1, gx, dinv, BW1, b1t, BW2, g2_o):
  p = dinv[...] * (a0[...] + a1[...] + gx[...])
  h1 = jnp.maximum(
      jnp.dot(p, BW1[...], preferred_element_type=jnp.float32) + b1t[...],
      0.0)
  g2_o[...] = dinv[...] * jnp.dot(h1, BW2[...],
                                  preferred_element_type=jnp.float32)


def _tc2(a0, a1, gx, dinv, BW1, b1t, BW2):
  return pl.pallas_call(
      _tc2_body,
      grid=(_TCG,),
      in_specs=[_vspec(), _vspec(), _vspec(), _vspec(),
                _full_spec(128, 256), _full_spec(1, 256),
                _full_spec(256, 128)],
      out_specs=_vspec(),
      out_shape=_f32(VR, 128),
  )(a0, a1, gx, dinv, BW1, b1t, BW2)


def _tc3_body(a0, a1, g2, dinv, b2t, BW3, g3_o):
  h2 = jnp.maximum(dinv[...] * (a0[...] + a1[...] + g2[...]) + b2t[...], 0.0)
  g3_o[...] = dinv[...] * jnp.dot(h2, BW3[...],
                                  preferred_element_type=jnp.float32)


def _tc3(a0, a1, g2, dinv, b2t, BW3):
  return pl.pallas_call(
      _tc3_body,
      grid=(_TCG,),
      in_specs=[_vspec(), _vspec(), _vspec(), _vspec(),
                _full_spec(1, 128), _full_spec(128, 128)],
      out_specs=_vspec(),
      out_shape=_f32(VR, 128),
  )(a0, a1, g2, dinv, b2t, BW3)


def _tc4_body(a0, a1, g3, dinv, b3t, e8t, h_o):
  h3 = jnp.maximum(dinv[...] * (a0[...] + a1[...] + g3[...]) + b3t[...], 0.0)
  h_o[...] = h3 + e8t[...]


def _tc4(a0, a1, g3, dinv, b3t, e8t):
  return pl.pallas_call(
      _tc4_body,
      grid=(_TCG,),
      in_specs=[_vspec(), _vspec(), _vspec(), _vspec(),
                _full_spec(1, 128), _full_spec(1, 128)],
      out_specs=_vspec(),
      out_shape=_f32(VR, 128),
  )(a0, a1, g3, dinv, b3t, e8t)


def _tc5_body(s0, s1, Wfc, bfc, out_o):
  acc = (s0[...] + s1[...])[:G]
  sums = acc[:, :8]
  cnts = jnp.maximum(acc[:, 8:9], 1.0)
  pooled = sums / cnts
  out_o[...] = jnp.dot(pooled, Wfc[...],
                       preferred_element_type=jnp.float32) + bfc[...]


def _tc5(s0, s1, Wfc, bfc):
  return pl.pallas_call(
      _tc5_body,
      out_shape=_f32(G, 3),
  )(s0, s1, Wfc, bfc)


def _view(a):
  return a.reshape(VR, 128)


def _unview(a):
  return a.reshape(NP, 16)


@jax.jit
def kernel(x, edge_index, batch, W1, b1, W2, b2, W3, b3, Wfc, bfc):
  src = edge_index[0]
  dst = edge_index[1]
  f32 = jnp.float32
  x16 = jnp.pad(x, ((0, PAD), (0, 13)))
  batch_p = jnp.pad(batch, (0, PAD), constant_values=G)

  eye8 = jnp.eye(8, dtype=f32)
  BW1 = jnp.kron(eye8, jnp.pad(W1, ((0, 13), (0, 0))))
  BW2 = jnp.kron(eye8, W2)
  BW3 = jnp.kron(eye8, jnp.pad(W3, ((0, 0), (0, 8))))
  b1t = jnp.tile(b1, 8).reshape(1, 256)
  b2t = jnp.tile(b2, 8).reshape(1, 128)
  b3t = jnp.tile(jnp.pad(b3, (0, 8)), 8).reshape(1, 128)
  e8t = jnp.tile(jnp.zeros((16,), f32).at[8].set(1.0), 8).reshape(1, 128)

  ones_eb = jnp.ones((EB, 16), f32)
  zeros16 = jnp.zeros((NP, 16), f32)
  zgp = jnp.zeros((GP, 16), f32)

  deg_pp = _sc_deg(dst, ones_eb, zeros16)
  dinv, gx = _tc1(_view(deg_pp[0]), _view(deg_pp[1]), _view(x16))

  aggx = _prop(src, dst, _unview(gx), zeros16)
  g2 = _tc2(_view(aggx[0]), _view(aggx[1]), gx, dinv, BW1, b1t, BW2)

  agg2 = _prop(src, dst, _unview(g2), zeros16)
  g3 = _tc3(_view(agg2[0]), _view(agg2[1]), g2, dinv, b2t, BW3)

  agg3 = _prop(src, dst, _unview(g3), zeros16)
  h16 = _tc4(_view(agg3[0]), _view(agg3[1]), g3, dinv, b3t, e8t)

  sums_pp = _sc_pool(_unview(h16), batch_p, zgp)
  out = _tc5(sums_pp[0], sums_pp[1], Wfc, bfc.reshape(1, -1))
  return out


# speedup vs baseline: 2.3279x; 1.9119x over previous
import functools
import jax
import jax.numpy as jnp
from jax import lax
from jax.experimental import pallas as pl
from jax.experimental.pallas import tpu as pltpu
from jax.experimental.pallas import tpu_sc as plsc

N = 100000
E = 6400000
G = 128

NC = 2
NS = 16
NW = NC * NS

NP = 100096
PAD = NP - N
RPW = NP // NW
RPS = NP // NS
GP = 136

EPC = E // NC
EPT = EPC // NS

EB = 2000
PB8 = 2000
PB16 = 800

VN = NP // 16

_mesh = plsc.VectorSubcoreMesh(core_axis_name="c", subcore_axis_name="s")
_sc_params = pltpu.CompilerParams(use_tc_tiling_on_sc=False)


def _f32(*shape):
  return jax.ShapeDtypeStruct(shape, jnp.float32)


@functools.partial(
    pl.kernel,
    out_type=_f32(NC, NP, 8),
    mesh=_mesh,
    compiler_params=_sc_params,
    scratch_types=[
        pltpu.VMEM((EB,), jnp.int32),
        pltpu.VMEM((EB,), jnp.int32),
        pltpu.VMEM((EB, 8), jnp.float32),
        pltpu.SemaphoreType.DMA,
        pltpu.SemaphoreType.DMA,
        pltpu.VMEM_SHARED((NP, 8), jnp.float32),
    ],
)
def _sc_deg(dst_hbm, ones_hbm, zeros_hbm, out_hbm,
            idx0, idx1, ones_v, sem0, sem1, acc_sh):
  c = lax.axis_index("c")
  s = lax.axis_index("s")
  nstep = EPT // EB
  row0 = s * RPS
  pltpu.sync_copy(zeros_hbm.at[pl.ds(row0, RPS)], acc_sh.at[pl.ds(row0, RPS)])
  pltpu.sync_copy(ones_hbm, ones_v)
  plsc.subcore_barrier()
  ebase = c * EPC + s * EPT

  def _start(i, idx, sem):
    pltpu.async_copy(dst_hbm.at[pl.ds(ebase + i * EB, EB)], idx, sem)

  def _drain(i, idx, sem):
    pltpu.make_async_copy(dst_hbm.at[pl.ds(ebase + i * EB, EB)], idx,
                          sem).wait()
    pltpu.sync_copy(ones_v, acc_sh.at[idx], add=True)

  _start(0, idx0, sem0)

  def step(k, carry):
    i0 = 2 * k
    _start(i0 + 1, idx1, sem1)
    _drain(i0, idx0, sem0)

    @pl.when(i0 + 2 < nstep)
    def _():
      _start(i0 + 2, idx0, sem0)

    _drain(i0 + 1, idx1, sem1)
    return carry

  lax.fori_loop(0, nstep // 2, step, 0)
  plsc.subcore_barrier()
  pltpu.sync_copy(acc_sh.at[pl.ds(row0, RPS)],
                  out_hbm.at[c, pl.ds(row0, RPS)])


def _make_prop(F, pb):
  nstep = EPT // pb
  assert nstep * pb == EPT and nstep % 2 == 0 and pb % 8 == 0 and F % 8 == 0

  @functools.partial(
      pl.kernel,
      out_type=_f32(NC, NP, F),
      mesh=_mesh,
      compiler_params=_sc_params,
      scratch_types=[
          pltpu.VMEM((pb,), jnp.int32),
          pltpu.VMEM((pb,), jnp.int32),
          pltpu.VMEM((pb,), jnp.int32),
          pltpu.VMEM((pb,), jnp.int32),
          pltpu.VMEM((pb, F), jnp.float32),
          pltpu.VMEM((pb, F), jnp.float32),
          pltpu.SemaphoreType.DMA,
          pltpu.SemaphoreType.DMA,
          pltpu.VMEM_SHARED((NP, F), jnp.float32),
      ],
  )
  def _prop(src_hbm, dst_hbm, g_hbm, zeros_hbm, out_hbm,
            ixs0, ixd0, ixs1, ixd1, rows0, rows1, sem0, sem1, acc_sh):
    c = lax.axis_index("c")
    s = lax.axis_index("s")
    row0 = s * RPS
    pltpu.sync_copy(zeros_hbm.at[pl.ds(row0, RPS)],
                    acc_sh.at[pl.ds(row0, RPS)])
    plsc.subcore_barrier()
    ebase = c * EPC + s * EPT

    def _start(i, ixs, ixd, rows, sem):
      base = ebase + i * pb
      pltpu.sync_copy(src_hbm.at[pl.ds(base, pb)], ixs)
      pltpu.sync_copy(dst_hbm.at[pl.ds(base, pb)], ixd)
      pltpu.async_copy(g_hbm.at[ixs], rows, sem)

    def _drain(ixs, ixd, rows, sem):
      pltpu.make_async_copy(g_hbm.at[ixs], rows, sem).wait()
      pltpu.sync_copy(rows, acc_sh.at[ixd], add=True)

    _start(0, ixs0, ixd0, rows0, sem0)

    def step(k, carry):
      i0 = 2 * k
      _start(i0 + 1, ixs1, ixd1, rows1, sem1)
      _drain(ixs0, ixd0, rows0, sem0)

      @pl.when(i0 + 2 < nstep)
      def _():
        _start(i0 + 2, ixs0, ixd0, rows0, sem0)

      _drain(ixs1, ixd1, rows1, sem1)
      return carry

    lax.fori_loop(0, nstep // 2, step, 0)
    plsc.subcore_barrier()
    pltpu.sync_copy(acc_sh.at[pl.ds(row0, RPS)],
                    out_hbm.at[c, pl.ds(row0, RPS)])

  return _prop


_prop8 = _make_prop(8, PB8)
_prop16 = _make_prop(16, PB16)


@functools.partial(
    pl.kernel,
    out_type=_f32(NC, GP, 16),
    mesh=_mesh,
    compiler_params=_sc_params,
    scratch_types=[
        pltpu.VMEM((RPW,), jnp.int32),
        pltpu.VMEM((RPW, 16), jnp.float32),
        pltpu.VMEM_SHARED((GP, 16), jnp.float32),
    ],
)
def _sc_pool(h_hbm, batch_hbm, zeros_hbm, out_hbm, idx_v, rows_v, acc_sh):
  c = lax.axis_index("c")
  s = lax.axis_index("s")

  @pl.when(s == 0)
  def _():
    pltpu.sync_copy(zeros_hbm, acc_sh)

  plsc.subcore_barrier()
  row0 = (c * NS + s) * RPW
  pltpu.sync_copy(h_hbm.at[pl.ds(row0, RPW)], rows_v)
  pltpu.sync_copy(batch_hbm.at[pl.ds(row0, RPW)], idx_v)
  pltpu.sync_copy(rows_v, acc_sh.at[idx_v], add=True)
  plsc.subcore_barrier()

  @pl.when(s == 0)
  def _():
    pltpu.sync_copy(acc_sh, out_hbm.at[c])


_TCR = 3128
_TCG = VN // _TCR


def _vspec(w):
  return pl.BlockSpec((_TCR, w), lambda i: (i, 0))


def _pspec(w):
  def mk(c):
    return pl.BlockSpec((1, _TCR, w), lambda i, _c=c: (_c, i, 0))
  return mk


def _full_spec(r, f):
  return pl.BlockSpec((r, f), lambda i: (0, 0))


def _tc1_body(d0, d1, x, dinv_o, gx_o):
  dinv = lax.rsqrt(d0[0] + d1[0] + 1.0)
  dinv_o[...] = dinv
  gx_o[...] = x[...] * dinv


def _tc1(dpp, x):
  return pl.pallas_call(
      _tc1_body,
      grid=(_TCG,),
      in_specs=[_pspec(128)(0), _pspec(128)(1), _vspec(128)],
      out_specs=[_vspec(128), _vspec(128)],
      out_shape=[_f32(VN, 128), _f32(VN, 128)],
  )(dpp, dpp, x)


def _tc2_body(a0, a1, gx, dinv, BW1, b1t, BW2, R16, g2_o):
  p = dinv[...] * (a0[0] + a1[0] + gx[...])
  h1 = jnp.maximum(
      jnp.dot(p, BW1[...], preferred_element_type=jnp.float32) + b1t[...],
      0.0)
  dinv16 = jnp.dot(dinv[...], R16[...], preferred_element_type=jnp.float32)
  g2_o[...] = dinv16 * jnp.dot(h1, BW2[...],
                               preferred_element_type=jnp.float32)


def _tc2(app, gx, dinv, BW1, b1t, BW2, R16):
  return pl.pallas_call(
      _tc2_body,
      grid=(_TCG,),
      in_specs=[_pspec(128)(0), _pspec(128)(1), _vspec(128), _vspec(128),
                _full_spec(128, 512), _full_spec(1, 512),
                _full_spec(512, 256), _full_spec(128, 256)],
      out_specs=_vspec(256),
      out_shape=_f32(VN, 256),
  )(app, app, gx, dinv, BW1, b1t, BW2, R16)


def _tc3_body(a0, a1, g2, dinv, R16, b2t, BW3, g3_o):
  dinv16 = jnp.dot(dinv[...], R16[...], preferred_element_type=jnp.float32)
  h2 = jnp.maximum(dinv16 * (a0[0] + a1[0] + g2[...]) + b2t[...], 0.0)
  g3_o[...] = dinv[...] * jnp.dot(h2, BW3[...],
                                  preferred_element_type=jnp.float32)


def _tc3(app, g2, dinv, R16, b2t, BW3):
  return pl.pallas_call(
      _tc3_body,
      grid=(_TCG,),
      in_specs=[_pspec(256)(0), _pspec(256)(1), _vspec(256), _vspec(128),
                _full_spec(128, 256), _full_spec(1, 256),
                _full_spec(256, 128)],
      out_specs=_vspec(128),
      out_shape=_f32(VN, 128),
  )(app, app, g2, dinv, R16, b2t, BW3)


def _tc4_body(a0, a1, g3, dinv, b3t, P16, e8t, h_o):
  h3 = jnp.maximum(dinv[...] * (a0[0] + a1[0] + g3[...]) + b3t[...], 0.0)
  h_o[...] = jnp.dot(h3, P16[...],
                     preferred_element_type=jnp.float32) + e8t[...]


def _tc4(app, g3, dinv, b3t, P16, e8t):
  return pl.pallas_call(
      _tc4_body,
      grid=(_TCG,),
      in_specs=[_pspec(128)(0), _pspec(128)(1), _vspec(128), _vspec(128),
                _full_spec(1, 128), _full_spec(128, 256),
                _full_spec(1, 256)],
      out_specs=_vspec(256),
      out_shape=_f32(VN, 256),
  )(app, app, g3, dinv, b3t, P16, e8t)


def _tc5_body(s0, s1, Wfc, bfc, out_o):
  acc = (s0[...] + s1[...])[:G]
  sums = acc[:, :8]
  cnts = jnp.maximum(acc[:, 8:9], 1.0)
  pooled = sums / cnts
  out_o[...] = jnp.dot(pooled, Wfc[...],
                       preferred_element_type=jnp.float32) + bfc[...]


def _tc5(s0, s1, Wfc, bfc):
  return pl.pallas_call(
      _tc5_body,
      out_shape=_f32(G, 3),
  )(s0, s1, Wfc, bfc)


@jax.jit
def kernel(x, edge_index, batch, W1, b1, W2, b2, W3, b3, Wfc, bfc):
  src = edge_index[0]
  dst = edge_index[1]
  f32 = jnp.float32
  x8 = jnp.pad(x, ((0, PAD), (0, 5)))
  batch_p = jnp.pad(batch, (0, PAD), constant_values=G)

  eye16 = jnp.eye(16, dtype=f32)
  BW1 = jnp.kron(eye16, jnp.pad(W1, ((0, 5), (0, 0))))
  BW2 = jnp.kron(eye16, W2)
  BW3 = jnp.kron(eye16, W3)
  R16 = jnp.kron(eye16, jnp.zeros((8, 16), f32).at[0, :].set(1.0))
  P16 = jnp.kron(eye16, jnp.concatenate(
      [jnp.eye(8, dtype=f32), jnp.zeros((8, 8), f32)], axis=1))
  b1t = jnp.tile(b1, 16).reshape(1, 512)
  b2t = jnp.tile(b2, 16).reshape(1, 256)
  b3t = jnp.tile(b3, 16).reshape(1, 128)
  e8t = jnp.tile(jnp.zeros((16,), f32).at[8].set(1.0), 16).reshape(1, 256)

  ones_eb = jnp.ones((EB, 8), f32)
  zeros8 = jnp.zeros((NP, 8), f32)
  zeros16 = jnp.zeros((NP, 16), f32)
  zgp = jnp.zeros((GP, 16), f32)

  deg_pp = _sc_deg(dst, ones_eb, zeros8)
  dinv, gx = _tc1(deg_pp.reshape(NC, VN, 128), x8.reshape(VN, 128))

  aggx = _prop8(src, dst, gx.reshape(NP, 8), zeros8)
  g2 = _tc2(aggx.reshape(NC, VN, 128), gx, dinv, BW1, b1t, BW2, R16)

  agg2 = _prop16(src, dst, g2.reshape(NP, 16), zeros16)
  g3 = _tc3(agg2.reshape(NC, VN, 256), g2, dinv, R16, b2t, BW3)

  agg3 = _prop8(src, dst, g3.reshape(NP, 8), zeros8)
  h16 = _tc4(agg3.reshape(NC, VN, 128), g3, dinv, b3t, P16, e8t)

  sums_pp = _sc_pool(h16.reshape(NP, 16), batch_p, zgp)
  out = _tc5(sums_pp[0], sums_pp[1], Wfc, bfc.reshape(1, -1))
  return out


# speedup vs baseline: 2.4904x; 1.0698x over previous
import functools
import jax
import jax.numpy as jnp
from jax import lax
from jax.experimental import pallas as pl
from jax.experimental.pallas import tpu as pltpu
from jax.experimental.pallas import tpu_sc as plsc

N = 100000
E = 6400000
G = 128

NC = 2
NS = 16
NW = NC * NS

NP = 100096
PAD = NP - N
RPW = NP // NW
RPS = NP // NS
GP = 136

EPC = E // NC
EPT = EPC // NS

EB = 2000
PB8 = 2000
PB16 = 400

VN = NP // 16

_mesh = plsc.VectorSubcoreMesh(core_axis_name="c", subcore_axis_name="s")
_sc_params = pltpu.CompilerParams(use_tc_tiling_on_sc=False)


def _f32(*shape):
  return jax.ShapeDtypeStruct(shape, jnp.float32)


@functools.partial(
    pl.kernel,
    out_type=_f32(NC, NP, 8),
    mesh=_mesh,
    compiler_params=_sc_params,
    scratch_types=[
        pltpu.VMEM((EB,), jnp.int32),
        pltpu.VMEM((EB,), jnp.int32),
        pltpu.VMEM((EB, 8), jnp.float32),
        pltpu.SemaphoreType.DMA,
        pltpu.SemaphoreType.DMA,
        pltpu.VMEM_SHARED((NP, 8), jnp.float32),
    ],
)
def _sc_deg(dst_hbm, ones_hbm, zeros_hbm, out_hbm,
            idx0, idx1, ones_v, sem0, sem1, acc_sh):
  c = lax.axis_index("c")
  s = lax.axis_index("s")
  nstep = EPT // EB
  row0 = s * RPS
  pltpu.sync_copy(zeros_hbm.at[pl.ds(row0, RPS)], acc_sh.at[pl.ds(row0, RPS)])
  pltpu.sync_copy(ones_hbm, ones_v)
  plsc.subcore_barrier()
  ebase = c * EPC + s * EPT

  def _start(i, idx, sem):
    pltpu.async_copy(dst_hbm.at[pl.ds(ebase + i * EB, EB)], idx, sem)

  def _drain(i, idx, sem):
    pltpu.make_async_copy(dst_hbm.at[pl.ds(ebase + i * EB, EB)], idx,
                          sem).wait()
    pltpu.sync_copy(ones_v, acc_sh.at[idx], add=True)

  _start(0, idx0, sem0)

  def step(k, carry):
    i0 = 2 * k
    _start(i0 + 1, idx1, sem1)
    _drain(i0, idx0, sem0)

    @pl.when(i0 + 2 < nstep)
    def _():
      _start(i0 + 2, idx0, sem0)

    _drain(i0 + 1, idx1, sem1)
    return carry

  lax.fori_loop(0, nstep // 2, step, 0)
  plsc.subcore_barrier()
  pltpu.sync_copy(acc_sh.at[pl.ds(row0, RPS)],
                  out_hbm.at[c, pl.ds(row0, RPS)])


def _make_prop(F, pb):
  nstep = EPT // pb
  assert nstep * pb == EPT and nstep % 4 == 0 and pb % 8 == 0 and F % 8 == 0

  @functools.partial(
      pl.kernel,
      out_type=_f32(NC, NP, F),
      mesh=_mesh,
      compiler_params=_sc_params,
      scratch_types=(
          [pltpu.VMEM((pb,), jnp.int32)] * 8 +
          [pltpu.VMEM((pb, F), jnp.float32)] * 2 +
          [pltpu.SemaphoreType.DMA] * 6 +
          [pltpu.VMEM_SHARED((NP, F), jnp.float32)]
      ),
  )
  def _prop(src_hbm, dst_hbm, g_hbm, zeros_hbm, out_hbm,
            ixs0, ixd0, ixs1, ixd1, ixs2, ixd2, ixs3, ixd3,
            rowsA, rowsB, is0, is1, is2, is3, gsA, gsB, acc_sh):
    c = lax.axis_index("c")
    s = lax.axis_index("s")
    row0 = s * RPS
    pltpu.sync_copy(zeros_hbm.at[pl.ds(row0, RPS)],
                    acc_sh.at[pl.ds(row0, RPS)])
    plsc.subcore_barrier()
    ebase = c * EPC + s * EPT
    IX = [(ixs0, ixd0, is0), (ixs1, ixd1, is1),
          (ixs2, ixd2, is2), (ixs3, ixd3, is3)]
    RW = [(rowsA, gsA), (rowsB, gsB)]

    def _load(i, j):
      ixs, ixd, isem = IX[j]
      base = ebase + i * pb
      pltpu.async_copy(src_hbm.at[pl.ds(base, pb)], ixs, isem)
      pltpu.async_copy(dst_hbm.at[pl.ds(base, pb)], ixd, isem)

    def _gather(i, j, r):
      ixs, ixd, isem = IX[j]
      rows, gsem = RW[r]
      base = ebase + i * pb
      pltpu.make_async_copy(src_hbm.at[pl.ds(base, pb)], ixs, isem).wait()
      pltpu.make_async_copy(dst_hbm.at[pl.ds(base, pb)], ixd, isem).wait()
      pltpu.async_copy(g_hbm.at[ixs], rows, gsem)

    def _scatter(j, r):
      ixs, ixd, isem = IX[j]
      rows, gsem = RW[r]
      pltpu.make_async_copy(g_hbm.at[ixs], rows, gsem).wait()
      pltpu.sync_copy(rows, acc_sh.at[ixd], add=True)

    _load(0, 0)
    _gather(0, 0, 0)
    _load(1, 1)

    def step(k, carry):
      i0 = 4 * k
      _gather(i0 + 1, 1, 1)
      _load(i0 + 2, 2)
      _scatter(0, 0)
      _gather(i0 + 2, 2, 0)
      _load(i0 + 3, 3)
      _scatter(1, 1)
      _gather(i0 + 3, 3, 1)

      @pl.when(i0 + 4 < nstep)
      def _():
        _load(i0 + 4, 0)

      _scatter(2, 0)

      @pl.when(i0 + 4 < nstep)
      def _():
        _gather(i0 + 4, 0, 0)

      @pl.when(i0 + 5 < nstep)
      def _():
        _load(i0 + 5, 1)

      _scatter(3, 1)
      return carry

    lax.fori_loop(0, nstep // 4, step, 0)
    plsc.subcore_barrier()
    pltpu.sync_copy(acc_sh.at[pl.ds(row0, RPS)],
                    out_hbm.at[c, pl.ds(row0, RPS)])

  return _prop


_prop8 = _make_prop(8, PB8)
_prop16 = _make_prop(16, PB16)


@functools.partial(
    pl.kernel,
    out_type=_f32(NC, GP, 16),
    mesh=_mesh,
    compiler_params=_sc_params,
    scratch_types=[
        pltpu.VMEM((RPW,), jnp.int32),
        pltpu.VMEM((RPW, 16), jnp.float32),
        pltpu.VMEM_SHARED((GP, 16), jnp.float32),
    ],
)
def _sc_pool(h_hbm, batch_hbm, zeros_hbm, out_hbm, idx_v, rows_v, acc_sh):
  c = lax.axis_index("c")
  s = lax.axis_index("s")

  @pl.when(s == 0)
  def _():
    pltpu.sync_copy(zeros_hbm, acc_sh)

  plsc.subcore_barrier()
  row0 = (c * NS + s) * RPW
  pltpu.sync_copy(h_hbm.at[pl.ds(row0, RPW)], rows_v)
  pltpu.sync_copy(batch_hbm.at[pl.ds(row0, RPW)], idx_v)
  pltpu.sync_copy(rows_v, acc_sh.at[idx_v], add=True)
  plsc.subcore_barrier()

  @pl.when(s == 0)
  def _():
    pltpu.sync_copy(acc_sh, out_hbm.at[c])


_TCR = 3128
_TCG = VN // _TCR


def _vspec(w):
  return pl.BlockSpec((_TCR, w), lambda i: (i, 0))


def _pspec(w):
  def mk(c):
    return pl.BlockSpec((1, _TCR, w), lambda i, _c=c: (_c, i, 0))
  return mk


def _full_spec(r, f):
  return pl.BlockSpec((r, f), lambda i: (0, 0))


def _tc1_body(d0, d1, x, dinv_o, gx_o):
  dinv = lax.rsqrt(d0[0] + d1[0] + 1.0)
  dinv_o[...] = dinv
  gx_o[...] = x[...] * dinv


def _tc1(dpp, x):
  return pl.pallas_call(
      _tc1_body,
      grid=(_TCG,),
      in_specs=[_pspec(128)(0), _pspec(128)(1), _vspec(128)],
      out_specs=[_vspec(128), _vspec(128)],
      out_shape=[_f32(VN, 128), _f32(VN, 128)],
  )(dpp, dpp, x)


def _tc2_body(a0, a1, gx, dinv, BW1, b1t, BW2, R16, g2_o):
  p = dinv[...] * (a0[0] + a1[0] + gx[...])
  h1 = jnp.maximum(
      jnp.dot(p, BW1[...], preferred_element_type=jnp.float32) + b1t[...],
      0.0)
  dinv16 = jnp.dot(dinv[...], R16[...], preferred_element_type=jnp.float32)
  g2_o[...] = dinv16 * jnp.dot(h1, BW2[...],
                               preferred_element_type=jnp.float32)


def _tc2(app, gx, dinv, BW1, b1t, BW2, R16):
  return pl.pallas_call(
      _tc2_body,
      grid=(_TCG,),
      in_specs=[_pspec(128)(0), _pspec(128)(1), _vspec(128), _vspec(128),
                _full_spec(128, 512), _full_spec(1, 512),
                _full_spec(512, 256), _full_spec(128, 256)],
      out_specs=_vspec(256),
      out_shape=_f32(VN, 256),
  )(app, app, gx, dinv, BW1, b1t, BW2, R16)


def _tc3_body(a0, a1, g2, dinv, R16, b2t, BW3, g3_o):
  dinv16 = jnp.dot(dinv[...], R16[...], preferred_element_type=jnp.float32)
  h2 = jnp.maximum(dinv16 * (a0[0] + a1[0] + g2[...]) + b2t[...], 0.0)
  g3_o[...] = dinv[...] * jnp.dot(h2, BW3[...],
                                  preferred_element_type=jnp.float32)


def _tc3(app, g2, dinv, R16, b2t, BW3):
  return pl.pallas_call(
      _tc3_body,
      grid=(_TCG,),
      in_specs=[_pspec(256)(0), _pspec(256)(1), _vspec(256), _vspec(128),
                _full_spec(128, 256), _full_spec(1, 256),
                _full_spec(256, 128)],
      out_specs=_vspec(128),
      out_shape=_f32(VN, 128),
  )(app, app, g2, dinv, R16, b2t, BW3)


def _tc4_body(a0, a1, g3, dinv, b3t, P16, e8t, h_o):
  h3 = jnp.maximum(dinv[...] * (a0[0] + a1[0] + g3[...]) + b3t[...], 0.0)
  h_o[...] = jnp.dot(h3, P16[...],
                     preferred_element_type=jnp.float32) + e8t[...]


def _tc4(app, g3, dinv, b3t, P16, e8t):
  return pl.pallas_call(
      _tc4_body,
      grid=(_TCG,),
      in_specs=[_pspec(128)(0), _pspec(128)(1), _vspec(128), _vspec(128),
                _full_spec(1, 128), _full_spec(128, 256),
                _full_spec(1, 256)],
      out_specs=_vspec(256),
      out_shape=_f32(VN, 256),
  )(app, app, g3, dinv, b3t, P16, e8t)


def _tc5_body(s0, s1, Wfc, bfc, out_o):
  acc = (s0[...] + s1[...])[:G]
  sums = acc[:, :8]
  cnts = jnp.maximum(acc[:, 8:9], 1.0)
  pooled = sums / cnts
  out_o[...] = jnp.dot(pooled, Wfc[...],
                       preferred_element_type=jnp.float32) + bfc[...]


def _tc5(s0, s1, Wfc, bfc):
  return pl.pallas_call(
      _tc5_body,
      out_shape=_f32(G, 3),
  )(s0, s1, Wfc, bfc)


@jax.jit
def kernel(x, edge_index, batch, W1, b1, W2, b2, W3, b3, Wfc, bfc):
  src = edge_index[0]
  dst = edge_index[1]
  f32 = jnp.float32
  x8 = jnp.pad(x, ((0, PAD), (0, 5)))
  batch_p = jnp.pad(batch, (0, PAD), constant_values=G)

  eye16 = jnp.eye(16, dtype=f32)
  BW1 = jnp.kron(eye16, jnp.pad(W1, ((0, 5), (0, 0))))
  BW2 = jnp.kron(eye16, W2)
  BW3 = jnp.kron(eye16, W3)
  R16 = jnp.kron(eye16, jnp.zeros((8, 16), f32).at[0, :].set(1.0))
  P16 = jnp.kron(eye16, jnp.concatenate(
      [jnp.eye(8, dtype=f32), jnp.zeros((8, 8), f32)], axis=1))
  b1t = jnp.tile(b1, 16).reshape(1, 512)
  b2t = jnp.tile(b2, 16).reshape(1, 256)
  b3t = jnp.tile(b3, 16).reshape(1, 128)
  e8t = jnp.tile(jnp.zeros((16,), f32).at[8].set(1.0), 16).reshape(1, 256)

  ones_eb = jnp.ones((EB, 8), f32)
  zeros8 = jnp.zeros((NP, 8), f32)
  zeros16 = jnp.zeros((NP, 16), f32)
  zgp = jnp.zeros((GP, 16), f32)

  deg_pp = _sc_deg(dst, ones_eb, zeros8)
  dinv, gx = _tc1(deg_pp.reshape(NC, VN, 128), x8.reshape(VN, 128))

  aggx = _prop8(src, dst, gx.reshape(NP, 8), zeros8)
  g2 = _tc2(aggx.reshape(NC, VN, 128), gx, dinv, BW1, b1t, BW2, R16)

  agg2 = _prop16(src, dst, g2.reshape(NP, 16), zeros16)
  g3 = _tc3(agg2.reshape(NC, VN, 256), g2, dinv, R16, b2t, BW3)

  agg3 = _prop8(src, dst, g3.reshape(NP, 8), zeros8)
  h16 = _tc4(agg3.reshape(NC, VN, 128), g3, dinv, b3t, P16, e8t)

  sums_pp = _sc_pool(h16.reshape(NP, 16), batch_p, zgp)
  out = _tc5(sums_pp[0], sums_pp[1], Wfc, bfc.reshape(1, -1))
  return out
